# baseline (speedup 1.0000x reference)
import functools

import jax
import jax.numpy as jnp
from jax import lax
from jax.experimental import pallas as pl
from jax.experimental.pallas import tpu as pltpu
from jax.experimental.pallas import tpu_sc as plsc

DIM = 1024
MEM_DIM = 128
TABLE = 100000
EPS = 1e-06
N_TOK = 2 * 4096

TN = 2048
GRID = N_TOK // TN


def _sc_gather(table, idx):
    info = plsc.get_sparse_core_info()
    nw = info.num_cores * info.num_subcores
    bpw = N_TOK // nw
    mesh = plsc.VectorSubcoreMesh(core_axis_name="c", subcore_axis_name="s")

    @functools.partial(
        pl.kernel,
        mesh=mesh,
        out_type=jax.ShapeDtypeStruct((N_TOK, MEM_DIM), jnp.float32),
        scratch_types=[
            pltpu.VMEM((bpw,), jnp.int32),
            pltpu.VMEM((bpw, MEM_DIM), jnp.float32),
            pltpu.SemaphoreType.DMA,
        ],
    )
    def gather_kernel(table_hbm, idx_hbm, out_hbm, idx_v, rows_v, sem):
        wid = lax.axis_index("s") * info.num_cores + lax.axis_index("c")
        base = wid * bpw
        pltpu.sync_copy(idx_hbm.at[pl.ds(base, bpw)], idx_v)
        pltpu.async_copy(table_hbm.at[idx_v], rows_v, sem).wait()
        pltpu.sync_copy(rows_v, out_hbm.at[pl.ds(base, bpw)])

    return gather_kernel(table, idx)


def _tc_body(ids_ref, x_ref, mem_ref, rmsw_ref, qw_ref, kw_ref, vw_ref,
             out_ref, w_scr, vwt_scr):
    @pl.when(pl.program_id(0) == 0)
    def _():
        w_scr[...] = lax.dot_general(
            qw_ref[...], kw_ref[...], (((0,), (0,)), ((), ())),
            preferred_element_type=jnp.float32)
        vwt_scr[...] = vw_ref[...].T

    x = x_ref[...]
    var = jnp.mean(x * x, axis=-1, keepdims=True)
    xn = x * lax.rsqrt(var + EPS) * rmsw_ref[...]
    p = jnp.dot(xn, w_scr[...], preferred_element_type=jnp.float32)
    mem = mem_ref[...]
    logit = jnp.sum(p * mem, axis=-1, keepdims=True) * (1.0 / 32.0)
    gate = jax.nn.sigmoid(logit)
    gate = gate * (ids_ref[...] != 0).astype(jnp.float32)
    v = jnp.dot(mem, vwt_scr[...], preferred_element_type=jnp.float32)
    out_ref[...] = gate * v


def _tc_call(ids_col, x2, mem, rms_w2, q_w, k_w, v_w):
    return pl.pallas_call(
        _tc_body,
        grid=(GRID,),
        in_specs=[
            pl.BlockSpec((TN, 1), lambda i: (i, 0)),
            pl.BlockSpec((TN, DIM), lambda i: (i, 0)),
            pl.BlockSpec((TN, MEM_DIM), lambda i: (i, 0)),
            pl.BlockSpec((1, DIM), lambda i: (0, 0)),
            pl.BlockSpec((DIM, DIM), lambda i: (0, 0)),
            pl.BlockSpec((DIM, MEM_DIM), lambda i: (0, 0)),
            pl.BlockSpec((DIM, MEM_DIM), lambda i: (0, 0)),
        ],
        out_specs=pl.BlockSpec((TN, DIM), lambda i: (i, 0)),
        out_shape=jax.ShapeDtypeStruct((N_TOK, DIM), jnp.float32),
        scratch_shapes=[pltpu.VMEM((DIM, MEM_DIM), jnp.float32),
                        pltpu.VMEM((MEM_DIM, DIM), jnp.float32)],
    )(ids_col, x2, mem, rms_w2, q_w, k_w, v_w)


def kernel(x, bigram_ids, embed, k_w, v_w, q_w, rms_w):
    ids_flat = bigram_ids.reshape(N_TOK).astype(jnp.int32)
    mem = _sc_gather(embed, ids_flat)
    out = _tc_call(ids_flat.reshape(N_TOK, 1), x.reshape(N_TOK, DIM), mem,
                   rms_w.reshape(1, DIM), q_w, k_w, v_w)
    return out.reshape(x.shape)


# speedup vs baseline: 1.0300x; 1.0300x over previous
import functools

import jax
import jax.numpy as jnp
from jax import lax
from jax.experimental import pallas as pl
from jax.experimental.pallas import tpu as pltpu
from jax.experimental.pallas import tpu_sc as plsc

DIM = 1024
MEM_DIM = 128
TABLE = 100000
EPS = 1e-06
N_TOK = 2 * 4096

TN = 1024
GRID = N_TOK // TN


def _sc_gather(table, idx):
    info = plsc.get_sparse_core_info()
    nw = info.num_cores * info.num_subcores
    bpw = N_TOK // nw
    mesh = plsc.VectorSubcoreMesh(core_axis_name="c", subcore_axis_name="s")

    @functools.partial(
        pl.kernel,
        mesh=mesh,
        out_type=jax.ShapeDtypeStruct((N_TOK, MEM_DIM), jnp.float32),
        scratch_types=[
            pltpu.VMEM((bpw,), jnp.int32),
            pltpu.VMEM((bpw, MEM_DIM), jnp.float32),
            pltpu.SemaphoreType.DMA,
        ],
    )
    def gather_kernel(table_hbm, idx_hbm, out_hbm, idx_v, rows_v, sem):
        wid = lax.axis_index("s") * info.num_cores + lax.axis_index("c")
        base = wid * bpw
        pltpu.sync_copy(idx_hbm.at[pl.ds(base, bpw)], idx_v)
        pltpu.async_copy(table_hbm.at[idx_v], rows_v, sem).wait()
        pltpu.sync_copy(rows_v, out_hbm.at[pl.ds(base, bpw)])

    return gather_kernel(table, idx)


def _tc_body(ids_ref, x_ref, mem_ref, rmsw_ref, qw_ref, kw_ref, vw_ref,
             out_ref, w_scr, vwt_scr):
    @pl.when(pl.program_id(0) == 0)
    def _():
        w_scr[...] = lax.dot_general(
            qw_ref[...], kw_ref[...], (((0,), (0,)), ((), ())),
            preferred_element_type=jnp.float32)
        vwt_scr[...] = vw_ref[...].T

    x = x_ref[...]
    var = jnp.mean(x * x, axis=-1, keepdims=True)
    xn = x * lax.rsqrt(var + EPS) * rmsw_ref[...]
    p = jnp.dot(xn.astype(jnp.bfloat16), w_scr[...].astype(jnp.bfloat16),
                preferred_element_type=jnp.float32)
    mem = mem_ref[...]
    logit = jnp.sum(p * mem, axis=-1, keepdims=True) * (1.0 / 32.0)
    gate = jax.nn.sigmoid(logit)
    gate = gate * (ids_ref[...] != 0).astype(jnp.float32)
    v = jnp.dot(mem.astype(jnp.bfloat16), vwt_scr[...].astype(jnp.bfloat16),
                preferred_element_type=jnp.float32)
    out_ref[...] = gate * v


def _tc_call(ids_col, x2, mem, rms_w2, q_w, k_w, v_w):
    return pl.pallas_call(
        _tc_body,
        grid=(GRID,),
        in_specs=[
            pl.BlockSpec((TN, 1), lambda i: (i, 0)),
            pl.BlockSpec((TN, DIM), lambda i: (i, 0)),
            pl.BlockSpec((TN, MEM_DIM), lambda i: (i, 0)),
            pl.BlockSpec((1, DIM), lambda i: (0, 0)),
            pl.BlockSpec((DIM, DIM), lambda i: (0, 0)),
            pl.BlockSpec((DIM, MEM_DIM), lambda i: (0, 0)),
            pl.BlockSpec((DIM, MEM_DIM), lambda i: (0, 0)),
        ],
        out_specs=pl.BlockSpec((TN, DIM), lambda i: (i, 0)),
        out_shape=jax.ShapeDtypeStruct((N_TOK, DIM), jnp.float32),
        scratch_shapes=[pltpu.VMEM((DIM, MEM_DIM), jnp.float32),
                        pltpu.VMEM((MEM_DIM, DIM), jnp.float32)],
    )(ids_col, x2, mem, rms_w2, q_w, k_w, v_w)


def kernel(x, bigram_ids, embed, k_w, v_w, q_w, rms_w):
    ids_flat = bigram_ids.reshape(N_TOK).astype(jnp.int32)
    mem = _sc_gather(embed, ids_flat)
    out = _tc_call(ids_flat.reshape(N_TOK, 1), x.reshape(N_TOK, DIM), mem,
                   rms_w.reshape(1, DIM), q_w, k_w, v_w)
    return out.reshape(x.shape)
